# baseline (speedup 1.0000x reference)
import functools

import jax
import jax.numpy as jnp
from jax import lax
from jax.experimental import pallas as pl
from jax.experimental.pallas import tpu as pltpu
from jax.experimental.pallas import tpu_sc as plsc

_NS = 16
_CH = 128
_SB = 8
_BN = 1000


def _tc_nbr_body(x_ref, wn_ref, oa_ref, ob_ref):
    hn = jnp.dot(x_ref[...], wn_ref[...], preferred_element_type=jnp.float32)
    oa_ref[...] = hn[:, :128]
    ob_ref[...] = hn[:, 128:]


def _tc_mid_body(x_ref, aa_ref, ab_ref, ws_ref, b_ref, g_ref, be_ref,
                 wn7_ref, h_ref, oa_ref, ob_ref):
    x = x_ref[...]
    agg = jnp.concatenate([aa_ref[...], ab_ref[...]], axis=1)
    c = jnp.dot(x, ws_ref[...], preferred_element_type=jnp.float32)
    c = c + agg + b_ref[...]
    mu = jnp.mean(c, axis=1, keepdims=True)
    var = jnp.mean((c - mu) ** 2, axis=1, keepdims=True)
    ln = (c - mu) * lax.rsqrt(var + 1e-5) * g_ref[...] + be_ref[...]
    h = x + jnp.maximum(ln, 0.0)
    h_ref[...] = h
    hn7 = jnp.dot(h, wn7_ref[...], preferred_element_type=jnp.float32)
    oa_ref[...] = hn7[:, :128]
    ob_ref[...] = hn7[:, 128:]


def _tc_out_body(h_ref, aa_ref, ab_ref, ws_ref, b_ref, g_ref, be_ref,
                 wp_ref, bp_ref, o_ref):
    h = h_ref[...]
    agg = jnp.concatenate([aa_ref[...], ab_ref[...]], axis=1)
    c = jnp.dot(h, ws_ref[...], preferred_element_type=jnp.float32)
    c = c + agg + b_ref[...]
    mu = jnp.mean(c, axis=1, keepdims=True)
    var = jnp.mean((c - mu) ** 2, axis=1, keepdims=True)
    ln = (c - mu) * lax.rsqrt(var + 1e-5) * g_ref[...] + be_ref[...]
    h2 = h + jnp.maximum(ln, 0.0)
    o_ref[...] = (jnp.dot(h2, wp_ref[...], preferred_element_type=jnp.float32)
                  + bp_ref[...])


def _row_spec(w):
    return pl.BlockSpec((_BN, w), lambda i: (i, 0))


def _full_spec(shape):
    return pl.BlockSpec(shape, lambda i: tuple(0 for _ in shape))


def _sc_agg_call(hn_a, hn_b, edges3, n_nodes):
    n_chunks = edges3.shape[1]
    n_acc = -(-n_nodes // (_NS * _CH)) * (_NS * _CH)
    npt = n_acc // _NS
    mesh = plsc.VectorSubcoreMesh(core_axis_name="c", subcore_axis_name="s")

    @functools.partial(
        pl.kernel,
        out_type=[jax.ShapeDtypeStruct((n_acc, 128), jnp.float32)] * 2,
        mesh=mesh,
        scratch_types=[
            pltpu.VMEM((_CH, 128), jnp.float32),
            pltpu.VMEM((_CH, 128), jnp.float32),
            pltpu.VMEM((_SB, 3, _CH), jnp.int32),
            pltpu.VMEM((_SB, 3, _CH), jnp.int32),
            pltpu.VMEM_SHARED((n_acc, 128), jnp.float32),
            pltpu.SemaphoreType.DMA,
            pltpu.SemaphoreType.DMA,
            pltpu.SemaphoreType.DMA,
            pltpu.SemaphoreType.DMA,
            pltpu.SemaphoreType.DMA,
        ],
    )
    def sc_kernel(hn_a_hbm, hn_b_hbm, edges_hbm,
                  agg_a_hbm, agg_b_hbm,
                  rows0_v, rows1_v, ib0, ib1, acc_sh,
                  gsem, ssem0, ssem1, is0, is1):
        rows_v = rows0_v
        c = lax.axis_index("c")
        s = lax.axis_index("s")
        base = s * npt
        sbufs = (ib0, ib1)
        isems = (is0, is1)

        def _zrow(i, _):
            for k in range(8):
                rows_v[i, pl.ds(k * 16, 16)] = jnp.zeros((16,), jnp.float32)
            return 0
        lax.fori_loop(0, _CH, _zrow, 0)
        for t in range(npt // _CH):
            pltpu.sync_copy(rows_v,
                            acc_sh.at[pl.ds(base + t * _CH, _CH)])
        plsc.subcore_barrier()

        def _edges(hn_hbm):
            bufs = (rows0_v, rows1_v)
            ssems = (ssem0, ssem1)
            ng = n_chunks // _SB
            ng2 = ng // 2

            def start_sidx(tg, p):
                pltpu.async_copy(edges_hbm.at[s].at[pl.ds(tg * _SB, _SB)],
                                 sbufs[p], isems[p])

            def wait_sidx(tg, p):
                pltpu.make_async_copy(
                    edges_hbm.at[s].at[pl.ds(tg * _SB, _SB)],
                    sbufs[p], isems[p]).wait()

            def gather(k, p, b):
                pltpu.async_copy(hn_hbm.at[sbufs[p].at[k].at[0]],
                                 bufs[b], gsem).wait()

            def start_scatter(k, p, b):
                pltpu.async_copy(bufs[b], acc_sh.at[sbufs[p].at[k].at[1]],
                                 ssems[b], add=True)

            def wait_scatter(k, p, b):
                pltpu.make_async_copy(
                    bufs[b], acc_sh.at[sbufs[p].at[k].at[1]], ssems[b]).wait()

            def mul(k, p, b):
                buf = bufs[b]
                ib = sbufs[p]

                def group_body(g, _):
                    ew16i = ib[k, 2, pl.ds(g * 16, 16)]
                    for e in range(16):
                        row = g * 16 + e
                        ewv = jnp.full(
                            (16,),
                            lax.bitcast_convert_type(ew16i[e], jnp.float32),
                            jnp.float32)
                        for kk in range(8):
                            sl = buf[row, pl.ds(kk * 16, 16)]
                            buf[row, pl.ds(kk * 16, 16)] = sl * ewv
                    return 0
                lax.fori_loop(0, _CH // 16, group_body, 0)

            start_sidx(0, 0)
            wait_sidx(0, 0)

            def pair_body(t2, _):
                for half in range(2):
                    tg = 2 * t2 + half
                    p = half
                    po = 1 - half
                    if half == 1:
                        wait_sidx(tg, p)
                    else:
                        @pl.when(t2 > 0)
                        def _():
                            wait_sidx(tg, p)
                    if half == 0:
                        start_sidx(tg + 1, po)
                    else:
                        @pl.when(t2 < ng2 - 1)
                        def _():
                            start_sidx(tg + 1, po)
                    for k in range(_SB):
                        gather(k, p, 0)
                        mul(k, p, 0)
                        pltpu.sync_copy(bufs[0],
                                        acc_sh.at[sbufs[p].at[k].at[1]],
                                        add=True)
                return 0
            lax.fori_loop(0, ng2, pair_body, 0)

        @pl.when(c == 0)
        def _():
            _edges(hn_a_hbm)

        @pl.when(c == 1)
        def _():
            _edges(hn_b_hbm)

        plsc.subcore_barrier()

        def _copy_out(agg_hbm):
            for t in range(npt // _CH):
                sl = pl.ds(base + t * _CH, _CH)
                pltpu.sync_copy(acc_sh.at[sl], rows_v)
                pltpu.sync_copy(rows_v, agg_hbm.at[sl])

        @pl.when(c == 0)
        def _():
            _copy_out(agg_a_hbm)

        @pl.when(c == 1)
        def _():
            _copy_out(agg_b_hbm)

    return sc_kernel(hn_a, hn_b, edges3)


def kernel(x, edge_index, edge_weight, W6_self, W6_nbr, b6, g6, beta6,
           W7_self, W7_nbr, b7, g7, beta7, Wp, bp):
    n, d = x.shape
    e = edge_weight.shape[0]
    out_d = Wp.shape[1]
    grid = (n // _BN,)

    n_chunks = (e + _NS * _CH - 1) // (_NS * _CH)
    n_chunks += (-n_chunks) % (2 * _SB)
    e_pad = _NS * n_chunks * _CH
    src = edge_index[0]
    dst = edge_index[1]
    ew = edge_weight
    if e_pad != e:
        p = e_pad - e
        src = jnp.concatenate([src, jnp.zeros((p,), jnp.int32)])
        dst = jnp.concatenate([dst, jnp.full((p,), n, jnp.int32)])
        ew = jnp.concatenate([ew, jnp.zeros((p,), jnp.float32)])
    edges3 = jnp.stack(
        [src.reshape(_NS, n_chunks, _CH),
         dst.reshape(_NS, n_chunks, _CH),
         lax.bitcast_convert_type(ew, jnp.int32).reshape(_NS, n_chunks, _CH)],
        axis=2)

    b6r, g6r, be6r = b6.reshape(1, d), g6.reshape(1, d), beta6.reshape(1, d)
    b7r, g7r, be7r = b7.reshape(1, d), g7.reshape(1, d), beta7.reshape(1, d)
    bpr = bp.reshape(1, out_d)

    tc_nbr = pl.pallas_call(
        _tc_nbr_body,
        grid=grid,
        in_specs=[_row_spec(d), _full_spec((d, d))],
        out_specs=[_row_spec(128), _row_spec(128)],
        out_shape=[jax.ShapeDtypeStruct((n, 128), jnp.float32)] * 2,
    )

    tc_mid = pl.pallas_call(
        _tc_mid_body,
        grid=grid,
        in_specs=[_row_spec(d), _row_spec(128), _row_spec(128),
                  _full_spec((d, d)), _full_spec((1, d)), _full_spec((1, d)),
                  _full_spec((1, d)), _full_spec((d, d))],
        out_specs=[_row_spec(d), _row_spec(128), _row_spec(128)],
        out_shape=[jax.ShapeDtypeStruct((n, d), jnp.float32),
                   jax.ShapeDtypeStruct((n, 128), jnp.float32),
                   jax.ShapeDtypeStruct((n, 128), jnp.float32)],
    )

    tc_out = pl.pallas_call(
        _tc_out_body,
        grid=grid,
        in_specs=[_row_spec(d), _row_spec(128), _row_spec(128),
                  _full_spec((d, d)), _full_spec((1, d)), _full_spec((1, d)),
                  _full_spec((1, d)), _full_spec((d, out_d)),
                  _full_spec((1, out_d))],
        out_specs=pl.BlockSpec((_BN, out_d), lambda i: (i, 0)),
        out_shape=jax.ShapeDtypeStruct((n, out_d), jnp.float32),
    )

    hn6a, hn6b = tc_nbr(x, W6_nbr)
    agg6a, agg6b = _sc_agg_call(hn6a, hn6b, edges3, n)
    h, hn7a, hn7b = tc_mid(x, agg6a, agg6b, W6_self, b6r, g6r, be6r, W7_nbr)
    agg7a, agg7b = _sc_agg_call(hn7a, hn7b, edges3, n)
    return tc_out(h, agg7a, agg7b, W7_self, b7r, g7r, be7r, Wp, bpr)


# speedup vs baseline: 1.4829x; 1.4829x over previous
import functools

import jax
import jax.numpy as jnp
from jax import lax
from jax.experimental import pallas as pl
from jax.experimental.pallas import tpu as pltpu
from jax.experimental.pallas import tpu_sc as plsc

_NS = 16
_CH = 128
_BN = 1000


def _tc_nbr_body(x_ref, wn_ref, oa_ref, ob_ref):
    hn = jnp.dot(x_ref[...], wn_ref[...], preferred_element_type=jnp.float32)
    oa_ref[...] = hn[:, :128]
    ob_ref[...] = hn[:, 128:]


def _tc_mid_body(x_ref, aa_ref, ab_ref, ws_ref, b_ref, g_ref, be_ref,
                 wn7_ref, h_ref, oa_ref, ob_ref):
    x = x_ref[...]
    agg = jnp.concatenate([aa_ref[...], ab_ref[...]], axis=1)
    c = jnp.dot(x, ws_ref[...], preferred_element_type=jnp.float32)
    c = c + agg + b_ref[...]
    mu = jnp.mean(c, axis=1, keepdims=True)
    var = jnp.mean((c - mu) ** 2, axis=1, keepdims=True)
    ln = (c - mu) * lax.rsqrt(var + 1e-5) * g_ref[...] + be_ref[...]
    h = x + jnp.maximum(ln, 0.0)
    h_ref[...] = h
    hn7 = jnp.dot(h, wn7_ref[...], preferred_element_type=jnp.float32)
    oa_ref[...] = hn7[:, :128]
    ob_ref[...] = hn7[:, 128:]


def _tc_out_body(h_ref, aa_ref, ab_ref, ws_ref, b_ref, g_ref, be_ref,
                 wp_ref, bp_ref, o_ref):
    h = h_ref[...]
    agg = jnp.concatenate([aa_ref[...], ab_ref[...]], axis=1)
    c = jnp.dot(h, ws_ref[...], preferred_element_type=jnp.float32)
    c = c + agg + b_ref[...]
    mu = jnp.mean(c, axis=1, keepdims=True)
    var = jnp.mean((c - mu) ** 2, axis=1, keepdims=True)
    ln = (c - mu) * lax.rsqrt(var + 1e-5) * g_ref[...] + be_ref[...]
    h2 = h + jnp.maximum(ln, 0.0)
    o_ref[...] = (jnp.dot(h2, wp_ref[...], preferred_element_type=jnp.float32)
                  + bp_ref[...])


def _row_spec(w):
    return pl.BlockSpec((_BN, w), lambda i: (i, 0))


def _full_spec(shape):
    return pl.BlockSpec(shape, lambda i: tuple(0 for _ in shape))


def _sc_agg_call(hn_a, hn_b, src3, dst3, ew3, n_nodes):
    n_chunks = src3.shape[1]
    n_acc = -(-n_nodes // (_NS * _CH)) * (_NS * _CH)
    npt = n_acc // _NS
    mesh = plsc.VectorSubcoreMesh(core_axis_name="c", subcore_axis_name="s")

    @functools.partial(
        pl.kernel,
        out_type=[jax.ShapeDtypeStruct((n_acc, 128), jnp.float32)] * 2,
        mesh=mesh,
        scratch_types=[
            pltpu.VMEM((n_chunks, _CH), jnp.int32),
            pltpu.VMEM((n_chunks, _CH), jnp.int32),
            pltpu.VMEM((n_chunks, _CH), jnp.float32),
            pltpu.VMEM((_CH, 128), jnp.float32),
            pltpu.VMEM_SHARED((n_acc, 128), jnp.float32),
            pltpu.SemaphoreType.DMA,
        ],
    )
    def sc_kernel(hn_a_hbm, hn_b_hbm, src_hbm, dst_hbm, ew_hbm,
                  agg_a_hbm, agg_b_hbm,
                  src_v, dst_v, ew_v, rows_v, acc_sh, sem):
        c = lax.axis_index("c")
        s = lax.axis_index("s")
        base = s * npt

        pltpu.sync_copy(src_hbm.at[s], src_v)
        pltpu.sync_copy(dst_hbm.at[s], dst_v)
        pltpu.sync_copy(ew_hbm.at[s], ew_v)

        def _zrow(i, _):
            for k in range(8):
                rows_v[i, pl.ds(k * 16, 16)] = jnp.zeros((16,), jnp.float32)
            return 0
        lax.fori_loop(0, _CH, _zrow, 0)
        for t in range(npt // _CH):
            pltpu.sync_copy(rows_v,
                            acc_sh.at[pl.ds(base + t * _CH, _CH)])
        plsc.subcore_barrier()

        def _edges(hn_hbm):
            def chunk_body(j, _):
                pltpu.async_copy(hn_hbm.at[src_v.at[j]], rows_v, sem).wait()

                def group_body(g, _):
                    ew16 = ew_v[j, pl.ds(g * 16, 16)]
                    for e in range(16):
                        row = g * 16 + e
                        ewv = jnp.full((16,), ew16[e], jnp.float32)
                        for k in range(8):
                            sl = rows_v[row, pl.ds(k * 16, 16)]
                            rows_v[row, pl.ds(k * 16, 16)] = sl * ewv
                    return 0
                lax.fori_loop(0, _CH // 16, group_body, 0)
                pltpu.sync_copy(rows_v, acc_sh.at[dst_v.at[j]], add=True)
                return 0
            lax.fori_loop(0, n_chunks, chunk_body, 0)

        @pl.when(c == 0)
        def _():
            _edges(hn_a_hbm)

        @pl.when(c == 1)
        def _():
            _edges(hn_b_hbm)

        plsc.subcore_barrier()

        def _copy_out(agg_hbm):
            for t in range(npt // _CH):
                sl = pl.ds(base + t * _CH, _CH)
                pltpu.sync_copy(acc_sh.at[sl], rows_v)
                pltpu.sync_copy(rows_v, agg_hbm.at[sl])

        @pl.when(c == 0)
        def _():
            _copy_out(agg_a_hbm)

        @pl.when(c == 1)
        def _():
            _copy_out(agg_b_hbm)

    return sc_kernel(hn_a, hn_b, src3, dst3, ew3)


def kernel(x, edge_index, edge_weight, W6_self, W6_nbr, b6, g6, beta6,
           W7_self, W7_nbr, b7, g7, beta7, Wp, bp):
    n, d = x.shape
    e = edge_weight.shape[0]
    out_d = Wp.shape[1]
    grid = (n // _BN,)

    n_chunks = (e + _NS * _CH - 1) // (_NS * _CH)
    e_pad = _NS * n_chunks * _CH
    src = edge_index[0]
    dst = edge_index[1]
    ew = edge_weight
    if e_pad != e:
        p = e_pad - e
        src = jnp.concatenate([src, jnp.zeros((p,), jnp.int32)])
        dst = jnp.concatenate([dst, jnp.full((p,), n, jnp.int32)])
        ew = jnp.concatenate([ew, jnp.zeros((p,), jnp.float32)])
    src3 = src.reshape(_NS, n_chunks, _CH)
    dst3 = dst.reshape(_NS, n_chunks, _CH)
    ew3 = ew.reshape(_NS, n_chunks, _CH)

    b6r, g6r, be6r = b6.reshape(1, d), g6.reshape(1, d), beta6.reshape(1, d)
    b7r, g7r, be7r = b7.reshape(1, d), g7.reshape(1, d), beta7.reshape(1, d)
    bpr = bp.reshape(1, out_d)

    tc_nbr = pl.pallas_call(
        _tc_nbr_body,
        grid=grid,
        in_specs=[_row_spec(d), _full_spec((d, d))],
        out_specs=[_row_spec(128), _row_spec(128)],
        out_shape=[jax.ShapeDtypeStruct((n, 128), jnp.float32)] * 2,
    )

    tc_mid = pl.pallas_call(
        _tc_mid_body,
        grid=grid,
        in_specs=[_row_spec(d), _row_spec(128), _row_spec(128),
                  _full_spec((d, d)), _full_spec((1, d)), _full_spec((1, d)),
                  _full_spec((1, d)), _full_spec((d, d))],
        out_specs=[_row_spec(d), _row_spec(128), _row_spec(128)],
        out_shape=[jax.ShapeDtypeStruct((n, d), jnp.float32),
                   jax.ShapeDtypeStruct((n, 128), jnp.float32),
                   jax.ShapeDtypeStruct((n, 128), jnp.float32)],
    )

    tc_out = pl.pallas_call(
        _tc_out_body,
        grid=grid,
        in_specs=[_row_spec(d), _row_spec(128), _row_spec(128),
                  _full_spec((d, d)), _full_spec((1, d)), _full_spec((1, d)),
                  _full_spec((1, d)), _full_spec((d, out_d)),
                  _full_spec((1, out_d))],
        out_specs=pl.BlockSpec((_BN, out_d), lambda i: (i, 0)),
        out_shape=jax.ShapeDtypeStruct((n, out_d), jnp.float32),
    )

    hn6a, hn6b = tc_nbr(x, W6_nbr)
    agg6a, agg6b = _sc_agg_call(hn6a, hn6b, src3, dst3, ew3, n)
    h, hn7a, hn7b = tc_mid(x, agg6a, agg6b, W6_self, b6r, g6r, be6r, W7_nbr)
    agg7a, agg7b = _sc_agg_call(hn7a, hn7b, src3, dst3, ew3, n)
    return tc_out(h, agg7a, agg7b, W7_self, b7r, g7r, be7r, Wp, bpr)


# speedup vs baseline: 1.7443x; 1.1763x over previous
import functools

import jax
import jax.numpy as jnp
from jax import lax
from jax.experimental import pallas as pl
from jax.experimental.pallas import tpu as pltpu
from jax.experimental.pallas import tpu_sc as plsc

_NS = 16
_CH = 128
_HC = 64
_BN = 1000


def _tc_nbr_body(x_ref, wn_ref, oa_ref, ob_ref):
    hn = jnp.dot(x_ref[...], wn_ref[...], preferred_element_type=jnp.float32)
    oa_ref[...] = hn[:, :128]
    ob_ref[...] = hn[:, 128:]


def _tc_mid_body(x_ref, aa_ref, ab_ref, ws_ref, b_ref, g_ref, be_ref,
                 wn7_ref, h_ref, oa_ref, ob_ref):
    x = x_ref[...]
    agg = jnp.concatenate([aa_ref[...], ab_ref[...]], axis=1)
    c = jnp.dot(x, ws_ref[...], preferred_element_type=jnp.float32)
    c = c + agg + b_ref[...]
    mu = jnp.mean(c, axis=1, keepdims=True)
    var = jnp.mean((c - mu) ** 2, axis=1, keepdims=True)
    ln = (c - mu) * lax.rsqrt(var + 1e-5) * g_ref[...] + be_ref[...]
    h = x + jnp.maximum(ln, 0.0)
    h_ref[...] = h
    hn7 = jnp.dot(h, wn7_ref[...], preferred_element_type=jnp.float32)
    oa_ref[...] = hn7[:, :128]
    ob_ref[...] = hn7[:, 128:]


def _tc_out_body(h_ref, aa_ref, ab_ref, ws_ref, b_ref, g_ref, be_ref,
                 wp_ref, bp_ref, o_ref):
    h = h_ref[...]
    agg = jnp.concatenate([aa_ref[...], ab_ref[...]], axis=1)
    c = jnp.dot(h, ws_ref[...], preferred_element_type=jnp.float32)
    c = c + agg + b_ref[...]
    mu = jnp.mean(c, axis=1, keepdims=True)
    var = jnp.mean((c - mu) ** 2, axis=1, keepdims=True)
    ln = (c - mu) * lax.rsqrt(var + 1e-5) * g_ref[...] + be_ref[...]
    h2 = h + jnp.maximum(ln, 0.0)
    o_ref[...] = (jnp.dot(h2, wp_ref[...], preferred_element_type=jnp.float32)
                  + bp_ref[...])


def _row_spec(w):
    return pl.BlockSpec((_BN, w), lambda i: (i, 0))


def _full_spec(shape):
    return pl.BlockSpec(shape, lambda i: tuple(0 for _ in shape))


def _sc_agg_call(hn_a, hn_b, src3, dst3, ew3, n_nodes):
    n_chunks = src3.shape[1]
    n_acc = -(-n_nodes // (_NS * _CH)) * (_NS * _CH)
    npt = n_acc // _NS
    mesh = plsc.VectorSubcoreMesh(core_axis_name="c", subcore_axis_name="s")

    @functools.partial(
        pl.kernel,
        out_type=[jax.ShapeDtypeStruct((n_acc, 128), jnp.float32)] * 2,
        mesh=mesh,
        scratch_types=[
            pltpu.VMEM((n_chunks, _CH), jnp.int32),
            pltpu.VMEM((n_chunks, _CH), jnp.int32),
            pltpu.VMEM((n_chunks, _CH), jnp.float32),
            pltpu.VMEM((2, _HC, 128), jnp.float32),
            pltpu.VMEM_SHARED((n_acc, 128), jnp.float32),
            pltpu.SemaphoreType.DMA,
            pltpu.SemaphoreType.DMA,
            pltpu.SemaphoreType.DMA,
            pltpu.SemaphoreType.DMA,
        ],
    )
    def sc_kernel(hn_a_hbm, hn_b_hbm, src_hbm, dst_hbm, ew_hbm,
                  agg_a_hbm, agg_b_hbm,
                  src_v, dst_v, ew_v, rows_v, acc_sh,
                  gs0, gs1, ss0, ss1):
        c = lax.axis_index("c")
        s = lax.axis_index("s")
        base = s * npt
        gsems = (gs0, gs1)
        ssems = (ss0, ss1)

        pltpu.sync_copy(src_hbm.at[s], src_v)
        pltpu.sync_copy(dst_hbm.at[s], dst_v)
        pltpu.sync_copy(ew_hbm.at[s], ew_v)

        def _zrow(i, _):
            for k in range(8):
                rows_v[0, i, pl.ds(k * 16, 16)] = jnp.zeros((16,), jnp.float32)
            return 0
        lax.fori_loop(0, _HC, _zrow, 0)
        for t in range(npt // _HC):
            pltpu.sync_copy(rows_v.at[0],
                            acc_sh.at[pl.ds(base + t * _HC, _HC)])
        plsc.subcore_barrier()

        def _edges(hn_hbm):
            def _islice(r, h):
                return (r, pl.ds(h * _HC, _HC))

            def start_gather(r, h):
                pltpu.async_copy(hn_hbm.at[src_v.at[_islice(r, h)]],
                                 rows_v.at[h], gsems[h])

            def wait_gather(r, h):
                pltpu.make_async_copy(hn_hbm.at[src_v.at[_islice(r, h)]],
                                      rows_v.at[h], gsems[h]).wait()

            def start_scatter(r, h):
                pltpu.async_copy(rows_v.at[h], acc_sh.at[dst_v.at[_islice(r, h)]],
                                 ssems[h], add=True)

            def wait_scatter(r, h):
                pltpu.make_async_copy(rows_v.at[h],
                                      acc_sh.at[dst_v.at[_islice(r, h)]],
                                      ssems[h]).wait()

            def mul(r, h):
                def group_body(g, _):
                    ew16 = ew_v[r, pl.ds(h * _HC + g * 16, 16)]
                    for e in range(16):
                        row = g * 16 + e
                        ewv = jnp.full((16,), ew16[e], jnp.float32)
                        for k in range(8):
                            sl = rows_v[h, row, pl.ds(k * 16, 16)]
                            rows_v[h, row, pl.ds(k * 16, 16)] = sl * ewv
                    return 0
                lax.fori_loop(0, _HC // 16, group_body, 0)

            start_gather(0, 0)

            def row_body(t, _):
                wait_gather(t, 0)

                @pl.when(t > 0)
                def _():
                    wait_scatter(t - 1, 1)
                start_gather(t, 1)
                mul(t, 0)
                start_scatter(t, 0)
                wait_gather(t, 1)

                @pl.when(t < n_chunks - 1)
                def _():
                    wait_scatter(t, 0)
                    start_gather(t + 1, 0)
                mul(t, 1)
                start_scatter(t, 1)
                return 0
            lax.fori_loop(0, n_chunks, row_body, 0)
            wait_scatter(n_chunks - 1, 0)
            wait_scatter(n_chunks - 1, 1)

        @pl.when(c == 0)
        def _():
            _edges(hn_a_hbm)

        @pl.when(c == 1)
        def _():
            _edges(hn_b_hbm)

        plsc.subcore_barrier()

        def _copy_out(agg_hbm):
            for t in range(npt // _HC):
                sl = pl.ds(base + t * _HC, _HC)
                pltpu.sync_copy(acc_sh.at[sl], rows_v.at[0])
                pltpu.sync_copy(rows_v.at[0], agg_hbm.at[sl])

        @pl.when(c == 0)
        def _():
            _copy_out(agg_a_hbm)

        @pl.when(c == 1)
        def _():
            _copy_out(agg_b_hbm)

    return sc_kernel(hn_a, hn_b, src3, dst3, ew3)


def kernel(x, edge_index, edge_weight, W6_self, W6_nbr, b6, g6, beta6,
           W7_self, W7_nbr, b7, g7, beta7, Wp, bp):
    n, d = x.shape
    e = edge_weight.shape[0]
    out_d = Wp.shape[1]
    grid = (n // _BN,)

    n_chunks = (e + _NS * _CH - 1) // (_NS * _CH)
    e_pad = _NS * n_chunks * _CH
    src = edge_index[0]
    dst = edge_index[1]
    ew = edge_weight
    if e_pad != e:
        p = e_pad - e
        src = jnp.concatenate([src, jnp.zeros((p,), jnp.int32)])
        dst = jnp.concatenate([dst, jnp.full((p,), n, jnp.int32)])
        ew = jnp.concatenate([ew, jnp.zeros((p,), jnp.float32)])
    src3 = src.reshape(_NS, n_chunks, _CH)
    dst3 = dst.reshape(_NS, n_chunks, _CH)
    ew3 = ew.reshape(_NS, n_chunks, _CH)

    b6r, g6r, be6r = b6.reshape(1, d), g6.reshape(1, d), beta6.reshape(1, d)
    b7r, g7r, be7r = b7.reshape(1, d), g7.reshape(1, d), beta7.reshape(1, d)
    bpr = bp.reshape(1, out_d)

    tc_nbr = pl.pallas_call(
        _tc_nbr_body,
        grid=grid,
        in_specs=[_row_spec(d), _full_spec((d, d))],
        out_specs=[_row_spec(128), _row_spec(128)],
        out_shape=[jax.ShapeDtypeStruct((n, 128), jnp.float32)] * 2,
    )

    tc_mid = pl.pallas_call(
        _tc_mid_body,
        grid=grid,
        in_specs=[_row_spec(d), _row_spec(128), _row_spec(128),
                  _full_spec((d, d)), _full_spec((1, d)), _full_spec((1, d)),
                  _full_spec((1, d)), _full_spec((d, d))],
        out_specs=[_row_spec(d), _row_spec(128), _row_spec(128)],
        out_shape=[jax.ShapeDtypeStruct((n, d), jnp.float32),
                   jax.ShapeDtypeStruct((n, 128), jnp.float32),
                   jax.ShapeDtypeStruct((n, 128), jnp.float32)],
    )

    tc_out = pl.pallas_call(
        _tc_out_body,
        grid=grid,
        in_specs=[_row_spec(d), _row_spec(128), _row_spec(128),
                  _full_spec((d, d)), _full_spec((1, d)), _full_spec((1, d)),
                  _full_spec((1, d)), _full_spec((d, out_d)),
                  _full_spec((1, out_d))],
        out_specs=pl.BlockSpec((_BN, out_d), lambda i: (i, 0)),
        out_shape=jax.ShapeDtypeStruct((n, out_d), jnp.float32),
    )

    hn6a, hn6b = tc_nbr(x, W6_nbr)
    agg6a, agg6b = _sc_agg_call(hn6a, hn6b, src3, dst3, ew3, n)
    h, hn7a, hn7b = tc_mid(x, agg6a, agg6b, W6_self, b6r, g6r, be6r, W7_nbr)
    agg7a, agg7b = _sc_agg_call(hn7a, hn7b, src3, dst3, ew3, n)
    return tc_out(h, agg7a, agg7b, W7_self, b7r, g7r, be7r, Wp, bpr)


# speedup vs baseline: 1.7522x; 1.0046x over previous
import functools

import jax
import jax.numpy as jnp
from jax import lax
from jax.experimental import pallas as pl
from jax.experimental.pallas import tpu as pltpu
from jax.experimental.pallas import tpu_sc as plsc

_NS = 16
_CH = 128
_HC = 64
_BN = 1000


def _tc_nbr_body(x_ref, wn_ref, oa_ref, ob_ref):
    hn = jnp.dot(x_ref[...], wn_ref[...], preferred_element_type=jnp.float32)
    oa_ref[...] = hn[:, :128]
    ob_ref[...] = hn[:, 128:]


def _tc_mid_body(x_ref, aa_ref, ab_ref, ws_ref, b_ref, g_ref, be_ref,
                 wn7_ref, h_ref, oa_ref, ob_ref):
    x = x_ref[...]
    agg = jnp.concatenate([aa_ref[...], ab_ref[...]], axis=1)
    c = jnp.dot(x, ws_ref[...], preferred_element_type=jnp.float32)
    c = c + agg + b_ref[...]
    mu = jnp.mean(c, axis=1, keepdims=True)
    var = jnp.mean((c - mu) ** 2, axis=1, keepdims=True)
    ln = (c - mu) * lax.rsqrt(var + 1e-5) * g_ref[...] + be_ref[...]
    h = x + jnp.maximum(ln, 0.0)
    h_ref[...] = h
    hn7 = jnp.dot(h, wn7_ref[...], preferred_element_type=jnp.float32)
    oa_ref[...] = hn7[:, :128]
    ob_ref[...] = hn7[:, 128:]


def _tc_out_body(h_ref, aa_ref, ab_ref, ws_ref, b_ref, g_ref, be_ref,
                 wp_ref, bp_ref, o_ref):
    h = h_ref[...]
    agg = jnp.concatenate([aa_ref[...], ab_ref[...]], axis=1)
    c = jnp.dot(h, ws_ref[...], preferred_element_type=jnp.float32)
    c = c + agg + b_ref[...]
    mu = jnp.mean(c, axis=1, keepdims=True)
    var = jnp.mean((c - mu) ** 2, axis=1, keepdims=True)
    ln = (c - mu) * lax.rsqrt(var + 1e-5) * g_ref[...] + be_ref[...]
    h2 = h + jnp.maximum(ln, 0.0)
    o_ref[...] = (jnp.dot(h2, wp_ref[...], preferred_element_type=jnp.float32)
                  + bp_ref[...])


def _row_spec(w):
    return pl.BlockSpec((_BN, w), lambda i: (i, 0))


def _full_spec(shape):
    return pl.BlockSpec(shape, lambda i: tuple(0 for _ in shape))


def _sc_agg_call(hn_a, hn_b, src3, dst3, ew3, n_nodes):
    n_chunks = src3.shape[1]
    n_acc = -(-n_nodes // (_NS * _CH)) * (_NS * _CH)
    npt = n_acc // _NS
    mesh = plsc.VectorSubcoreMesh(core_axis_name="c", subcore_axis_name="s")

    @functools.partial(
        pl.kernel,
        out_type=[jax.ShapeDtypeStruct((n_acc, 128), jnp.float32)] * 2,
        mesh=mesh,
        scratch_types=[
            pltpu.VMEM((n_chunks, _CH), jnp.int32),
            pltpu.VMEM((n_chunks, _CH), jnp.int32),
            pltpu.VMEM((n_chunks, _CH), jnp.float32),
            pltpu.VMEM((2, _HC, 128), jnp.float32),
            pltpu.VMEM_SHARED((n_acc, 128), jnp.float32),
            pltpu.SemaphoreType.DMA,
            pltpu.SemaphoreType.DMA,
            pltpu.SemaphoreType.DMA,
            pltpu.SemaphoreType.DMA,
        ],
    )
    def sc_kernel(hn_a_hbm, hn_b_hbm, src_hbm, dst_hbm, ew_hbm,
                  agg_a_hbm, agg_b_hbm,
                  src_v, dst_v, ew_v, rows_v, acc_sh,
                  gs0, gs1, ss0, ss1):
        c = lax.axis_index("c")
        s = lax.axis_index("s")
        base = s * npt
        gsems = (gs0, gs1)
        ssems = (ss0, ss1)

        pltpu.sync_copy(src_hbm.at[s], src_v)
        pltpu.sync_copy(dst_hbm.at[s], dst_v)
        pltpu.sync_copy(ew_hbm.at[s], ew_v)

        def _zrow(i, _):
            for k in range(8):
                rows_v[0, i, pl.ds(k * 16, 16)] = jnp.zeros((16,), jnp.float32)
            return 0
        lax.fori_loop(0, _HC, _zrow, 0)
        for t in range(npt // _HC):
            pltpu.sync_copy(rows_v.at[0],
                            acc_sh.at[pl.ds(base + t * _HC, _HC)])
        plsc.subcore_barrier()

        def _edges(hn_hbm):
            def _islice(r, h):
                return (r, pl.ds(h * _HC, _HC))

            def start_gather(r, h):
                pltpu.async_copy(hn_hbm.at[src_v.at[_islice(r, h)]],
                                 rows_v.at[h], gsems[h])

            def wait_gather(r, h):
                pltpu.make_async_copy(hn_hbm.at[src_v.at[_islice(r, h)]],
                                      rows_v.at[h], gsems[h]).wait()

            def start_scatter(r, h):
                pltpu.async_copy(rows_v.at[h], acc_sh.at[dst_v.at[_islice(r, h)]],
                                 ssems[h], add=True)

            def wait_scatter(r, h):
                pltpu.make_async_copy(rows_v.at[h],
                                      acc_sh.at[dst_v.at[_islice(r, h)]],
                                      ssems[h]).wait()

            def mul(r, h):
                def group_body(g, _):
                    ew16 = ew_v[r, pl.ds(h * _HC + g * 16, 16)]
                    for e in range(16):
                        row = g * 16 + e
                        ewv = jnp.full((16,), ew16[e], jnp.float32)
                        for k in range(8):
                            sl = rows_v[h, row, pl.ds(k * 16, 16)]
                            rows_v[h, row, pl.ds(k * 16, 16)] = sl * ewv
                    return 0
                lax.fori_loop(0, _HC // 16, group_body, 0)

            start_gather(0, 0)

            def row_body(t, _):
                wait_gather(t, 0)

                @pl.when(t > 0)
                def _():
                    wait_scatter(t - 1, 1)
                start_gather(t, 1)
                mul(t, 0)
                start_scatter(t, 0)
                wait_gather(t, 1)

                @pl.when(t < n_chunks - 1)
                def _():
                    wait_scatter(t, 0)
                    start_gather(t + 1, 0)
                mul(t, 1)
                start_scatter(t, 1)
                return 0
            lax.fori_loop(0, n_chunks, row_body, 0)
            wait_scatter(n_chunks - 1, 0)
            wait_scatter(n_chunks - 1, 1)

        @pl.when(c == 0)
        def _():
            _edges(hn_a_hbm)

        @pl.when(c == 1)
        def _():
            _edges(hn_b_hbm)

        plsc.subcore_barrier()

        def _copy_out(agg_hbm):
            sl = pl.ds(base, npt)
            pltpu.sync_copy(acc_sh.at[sl], agg_hbm.at[sl])

        @pl.when(c == 0)
        def _():
            _copy_out(agg_a_hbm)

        @pl.when(c == 1)
        def _():
            _copy_out(agg_b_hbm)

    return sc_kernel(hn_a, hn_b, src3, dst3, ew3)


def kernel(x, edge_index, edge_weight, W6_self, W6_nbr, b6, g6, beta6,
           W7_self, W7_nbr, b7, g7, beta7, Wp, bp):
    n, d = x.shape
    e = edge_weight.shape[0]
    out_d = Wp.shape[1]
    grid = (n // _BN,)

    n_chunks = (e + _NS * _CH - 1) // (_NS * _CH)
    e_pad = _NS * n_chunks * _CH
    src = edge_index[0]
    dst = edge_index[1]
    ew = edge_weight
    if e_pad != e:
        p = e_pad - e
        src = jnp.concatenate([src, jnp.zeros((p,), jnp.int32)])
        dst = jnp.concatenate([dst, jnp.full((p,), n, jnp.int32)])
        ew = jnp.concatenate([ew, jnp.zeros((p,), jnp.float32)])
    src3 = src.reshape(_NS, n_chunks, _CH)
    dst3 = dst.reshape(_NS, n_chunks, _CH)
    ew3 = ew.reshape(_NS, n_chunks, _CH)

    b6r, g6r, be6r = b6.reshape(1, d), g6.reshape(1, d), beta6.reshape(1, d)
    b7r, g7r, be7r = b7.reshape(1, d), g7.reshape(1, d), beta7.reshape(1, d)
    bpr = bp.reshape(1, out_d)

    tc_nbr = pl.pallas_call(
        _tc_nbr_body,
        grid=grid,
        in_specs=[_row_spec(d), _full_spec((d, d))],
        out_specs=[_row_spec(128), _row_spec(128)],
        out_shape=[jax.ShapeDtypeStruct((n, 128), jnp.float32)] * 2,
    )

    tc_mid = pl.pallas_call(
        _tc_mid_body,
        grid=grid,
        in_specs=[_row_spec(d), _row_spec(128), _row_spec(128),
                  _full_spec((d, d)), _full_spec((1, d)), _full_spec((1, d)),
                  _full_spec((1, d)), _full_spec((d, d))],
        out_specs=[_row_spec(d), _row_spec(128), _row_spec(128)],
        out_shape=[jax.ShapeDtypeStruct((n, d), jnp.float32),
                   jax.ShapeDtypeStruct((n, 128), jnp.float32),
                   jax.ShapeDtypeStruct((n, 128), jnp.float32)],
    )

    tc_out = pl.pallas_call(
        _tc_out_body,
        grid=grid,
        in_specs=[_row_spec(d), _row_spec(128), _row_spec(128),
                  _full_spec((d, d)), _full_spec((1, d)), _full_spec((1, d)),
                  _full_spec((1, d)), _full_spec((d, out_d)),
                  _full_spec((1, out_d))],
        out_specs=pl.BlockSpec((_BN, out_d), lambda i: (i, 0)),
        out_shape=jax.ShapeDtypeStruct((n, out_d), jnp.float32),
    )

    hn6a, hn6b = tc_nbr(x, W6_nbr)
    agg6a, agg6b = _sc_agg_call(hn6a, hn6b, src3, dst3, ew3, n)
    h, hn7a, hn7b = tc_mid(x, agg6a, agg6b, W6_self, b6r, g6r, be6r, W7_nbr)
    agg7a, agg7b = _sc_agg_call(hn7a, hn7b, src3, dst3, ew3, n)
    return tc_out(h, agg7a, agg7b, W7_self, b7r, g7r, be7r, Wp, bpr)
